# baseline (speedup 1.0000x reference)
import functools

import jax
import jax.numpy as jnp
from jax import lax
from jax.experimental import pallas as pl
from jax.experimental.pallas import tpu as pltpu
from jax.experimental.pallas import tpu_sc as plsc

N = 10000
E = 320000
R = 51
H = 32
NB = 4

NC, NS = 2, 16
NW = NC * NS
EW = E // NW
CH = 128
NFULL = EW // CH
TAIL = EW - NFULL * CH
RPT = 624
RTAIL = N - NS * RPT


def _proj_body(h_ref, w_ref, out_ref):
    out_ref[...] = jnp.dot(h_ref[...], w_ref[0],
                           preferred_element_type=jnp.float32)


def _proj(h, relw):
    din = h.shape[1]
    return pl.pallas_call(
        _proj_body,
        grid=(R,),
        in_specs=[
            pl.BlockSpec((N, din), lambda r: (0, 0)),
            pl.BlockSpec((1, din, H), lambda r: (r, 0, 0)),
        ],
        out_specs=pl.BlockSpec((N, H), lambda r: (r, 0)),
        out_shape=jax.ShapeDtypeStruct((R * N, H), jnp.float32),
    )(h, relw)


NBUF = 6
NSTEP = NFULL // NBUF


def _sc_body(proj_hbm, src_hbm, dst_hbm, ety_hbm, zeros_hbm, out_hbm,
             src_all, ety_all, key_ring, dst_ring, rows,
             src_t, dst_t, ety_t, key_t, rows_t,
             acc_sh, sem_g, sem_d, sem_s):
    c = lax.axis_index("c")
    s = lax.axis_index("s")
    wid = s * NC + c
    ebase = wid * EW

    pltpu.sync_copy(zeros_hbm.at[pl.ds(s * RPT, RPT)],
                    acc_sh.at[pl.ds(s * RPT, RPT)])

    @pl.when(s == NS - 1)
    def _():
        pltpu.sync_copy(zeros_hbm.at[pl.ds(NS * RPT, RTAIL)],
                        acc_sh.at[pl.ds(NS * RPT, RTAIL)])

    pltpu.sync_copy(src_hbm.at[pl.ds(ebase, EW)], src_all)
    pltpu.sync_copy(ety_hbm.at[pl.ds(ebase, EW)], ety_all)

    def fire(g, b):
        off = g * CH
        for k in range(CH // 16):
            sl = pl.ds(off + k * 16, 16)
            key_ring[b, pl.ds(k * 16, 16)] = ety_all[sl] * N + src_all[sl]
        pltpu.async_copy(dst_hbm.at[pl.ds(ebase + off, CH)], dst_ring.at[b],
                         sem_d[b])
        pltpu.async_copy(proj_hbm.at[key_ring.at[b]], rows.at[b], sem_g[b])

    for b in range(NBUF):
        fire(b, b)

    plsc.subcore_barrier()

    def body(i, carry):
        for b in range(NBUF):
            g = i * NBUF + b
            pltpu.make_async_copy(proj_hbm.at[pl.ds(0, CH)], rows.at[b],
                                  sem_g[b]).wait()
            pltpu.make_async_copy(dst_hbm.at[pl.ds(0, CH)], dst_ring.at[b],
                                  sem_d[b]).wait()
            pltpu.async_copy(rows.at[b], acc_sh.at[dst_ring.at[b]], sem_s[b],
                             add=True)
            pltpu.make_async_copy(proj_hbm.at[pl.ds(0, CH)], rows.at[b],
                                  sem_s[b]).wait()

            @pl.when(i < NSTEP - 1)
            def _():
                fire(g + NBUF, b)
        return carry

    lax.fori_loop(0, NSTEP, body, 0)

    tbase = ebase + NFULL * CH
    pltpu.sync_copy(src_hbm.at[pl.ds(tbase, TAIL)], src_t)
    pltpu.sync_copy(dst_hbm.at[pl.ds(tbase, TAIL)], dst_t)
    pltpu.sync_copy(ety_hbm.at[pl.ds(tbase, TAIL)], ety_t)
    key_t[...] = ety_t[...] * N + src_t[...]
    pltpu.async_copy(proj_hbm.at[key_t], rows_t, sem_g[0]).wait()
    pltpu.sync_copy(rows_t, acc_sh.at[dst_t], add=True)

    plsc.subcore_barrier()
    pltpu.sync_copy(acc_sh.at[pl.ds(s * RPT, RPT)],
                    out_hbm.at[pl.ds(c * N + s * RPT, RPT)])

    @pl.when(s == NS - 1)
    def _():
        pltpu.sync_copy(acc_sh.at[pl.ds(NS * RPT, RTAIL)],
                        out_hbm.at[pl.ds(c * N + NS * RPT, RTAIL)])


_sc_edge = pl.kernel(
    _sc_body,
    out_type=jax.ShapeDtypeStruct((NC * N, H), jnp.float32),
    mesh=plsc.VectorSubcoreMesh(core_axis_name="c", subcore_axis_name="s",
                                num_cores=NC, num_subcores=NS),
    scratch_types=[
        pltpu.VMEM((EW,), jnp.int32),
        pltpu.VMEM((EW,), jnp.int32),
        pltpu.VMEM((NBUF, CH), jnp.int32),
        pltpu.VMEM((NBUF, CH), jnp.int32),
        pltpu.VMEM((NBUF, CH, H), jnp.float32),
        pltpu.VMEM((TAIL,), jnp.int32),
        pltpu.VMEM((TAIL,), jnp.int32),
        pltpu.VMEM((TAIL,), jnp.int32),
        pltpu.VMEM((TAIL,), jnp.int32),
        pltpu.VMEM((TAIL, H), jnp.float32),
        pltpu.VMEM_SHARED((N, H), jnp.float32),
        [pltpu.SemaphoreType.DMA] * NBUF,
        [pltpu.SemaphoreType.DMA] * NBUF,
        [pltpu.SemaphoreType.DMA] * NBUF,
    ],
    compiler_params=pltpu.CompilerParams(use_tc_tiling_on_sc=False),
)


def _combine_body(h_ref, sl_ref, p_ref, out_ref):
    acc = jnp.dot(h_ref[...], sl_ref[...], preferred_element_type=jnp.float32)
    out_ref[...] = jnp.maximum(acc + p_ref[0:N, :] + p_ref[N:2 * N, :], 0.0)


def _combine(h, sl, part):
    return pl.pallas_call(
        _combine_body,
        out_shape=jax.ShapeDtypeStruct((N, H), jnp.float32),
    )(h, sl, part)


def kernel(x, bases0, wcomp0, sl0, bases1, wcomp1, sl1, edge_index, edge_type):
    src = edge_index[0]
    dst = edge_index[1]
    relw0 = jnp.einsum('rb,bio->rio', wcomp0, bases0)
    relw1 = jnp.einsum('rb,bio->rio', wcomp1, bases1)
    zeros = jnp.zeros((N, H), jnp.float32)

    h = x
    for relw, sl in ((relw0, sl0), (relw1, sl1)):
        proj = _proj(h, relw)
        part = _sc_edge(proj, src, dst, edge_type, zeros)
        h = _combine(h, sl, part)
    return h


# speedup vs baseline: 3.4727x; 3.4727x over previous
import functools

import jax
import jax.numpy as jnp
from jax import lax
from jax.experimental import pallas as pl
from jax.experimental.pallas import tpu as pltpu
from jax.experimental.pallas import tpu_sc as plsc

N = 10000
E = 320000
R = 51
H = 32
NB = 4

NC, NS = 2, 16
NW = NC * NS
EW = E // NW
CH = 128
NFULL = EW // CH
TAIL = EW - NFULL * CH
RPT = 624
RTAIL = N - NS * RPT


NPLANE = (R * H + 127) // 128


def _proj_body(h_ref, w_ref, out_ref):
    res = jnp.dot(h_ref[...], w_ref[...], preferred_element_type=jnp.float32)
    for j in range(NPLANE):
        out_ref[j] = res[:, j * 128:(j + 1) * 128]


def _proj(h, relw_pad):
    din = h.shape[1]
    tn = 1000
    return pl.pallas_call(
        _proj_body,
        grid=(N // tn,),
        in_specs=[
            pl.BlockSpec((tn, din), lambda t: (t, 0)),
            pl.BlockSpec((din, NPLANE * 128), lambda t: (0, 0)),
        ],
        out_specs=pl.BlockSpec((NPLANE, tn, 128), lambda t: (0, t, 0)),
        out_shape=jax.ShapeDtypeStruct((NPLANE, N, 128), jnp.float32),
    )(h, relw_pad)


NBUF = 6
NSTEP = NFULL // NBUF


def _sc_body(proj_hbm, src_hbm, dst_hbm, ety_hbm, zeros_hbm, out_hbm,
             src_all, ety_all, key_ring, dst_ring, rows,
             src_t, dst_t, ety_t, key_t, rows_t,
             acc_sh, sem_g, sem_d, sem_s):
    c = lax.axis_index("c")
    s = lax.axis_index("s")
    wid = s * NC + c
    ebase = wid * EW

    pltpu.sync_copy(zeros_hbm.at[pl.ds(s * RPT, RPT)],
                    acc_sh.at[pl.ds(s * RPT, RPT)])

    @pl.when(s == NS - 1)
    def _():
        pltpu.sync_copy(zeros_hbm.at[pl.ds(NS * RPT, RTAIL)],
                        acc_sh.at[pl.ds(NS * RPT, RTAIL)])

    pltpu.sync_copy(src_hbm.at[pl.ds(ebase, EW)], src_all)
    pltpu.sync_copy(ety_hbm.at[pl.ds(ebase, EW)], ety_all)

    def fire(g, b):
        off = g * CH
        for k in range(CH // 16):
            sl = pl.ds(off + k * 16, 16)
            ety = ety_all[sl]
            key_ring[b, pl.ds(k * 16, 16)] = (
                (ety >> 2) * (4 * N) + src_all[sl] * 4 + (ety & 3))
        pltpu.async_copy(dst_hbm.at[pl.ds(ebase + off, CH)], dst_ring.at[b],
                         sem_d[b])
        pltpu.async_copy(proj_hbm.at[key_ring.at[b]], rows.at[b], sem_g[b])

    for b in range(NBUF):
        fire(b, b)

    plsc.subcore_barrier()

    def body(i, carry):
        for b in range(NBUF):
            g = i * NBUF + b
            pltpu.make_async_copy(proj_hbm.at[pl.ds(0, CH)], rows.at[b],
                                  sem_g[b]).wait()
            pltpu.make_async_copy(dst_hbm.at[pl.ds(0, CH)], dst_ring.at[b],
                                  sem_d[b]).wait()
            pltpu.async_copy(rows.at[b], acc_sh.at[dst_ring.at[b]], sem_s[b],
                             add=True)
            pltpu.make_async_copy(proj_hbm.at[pl.ds(0, CH)], rows.at[b],
                                  sem_s[b]).wait()

            @pl.when(i < NSTEP - 1)
            def _():
                fire(g + NBUF, b)
        return carry

    lax.fori_loop(0, NSTEP, body, 0)

    tbase = ebase + NFULL * CH
    pltpu.sync_copy(src_hbm.at[pl.ds(tbase, TAIL)], src_t)
    pltpu.sync_copy(dst_hbm.at[pl.ds(tbase, TAIL)], dst_t)
    pltpu.sync_copy(ety_hbm.at[pl.ds(tbase, TAIL)], ety_t)
    ety = ety_t[...]
    key_t[...] = (ety >> 2) * (4 * N) + src_t[...] * 4 + (ety & 3)
    pltpu.async_copy(proj_hbm.at[key_t], rows_t, sem_g[0]).wait()
    pltpu.sync_copy(rows_t, acc_sh.at[dst_t], add=True)

    plsc.subcore_barrier()
    pltpu.sync_copy(acc_sh.at[pl.ds(s * RPT, RPT)],
                    out_hbm.at[pl.ds(c * N + s * RPT, RPT)])

    @pl.when(s == NS - 1)
    def _():
        pltpu.sync_copy(acc_sh.at[pl.ds(NS * RPT, RTAIL)],
                        out_hbm.at[pl.ds(c * N + NS * RPT, RTAIL)])


_sc_edge = pl.kernel(
    _sc_body,
    out_type=jax.ShapeDtypeStruct((NC * N, H), jnp.float32),
    mesh=plsc.VectorSubcoreMesh(core_axis_name="c", subcore_axis_name="s",
                                num_cores=NC, num_subcores=NS),
    scratch_types=[
        pltpu.VMEM((EW,), jnp.int32),
        pltpu.VMEM((EW,), jnp.int32),
        pltpu.VMEM((NBUF, CH), jnp.int32),
        pltpu.VMEM((NBUF, CH), jnp.int32),
        pltpu.VMEM((NBUF, CH, H), jnp.float32),
        pltpu.VMEM((TAIL,), jnp.int32),
        pltpu.VMEM((TAIL,), jnp.int32),
        pltpu.VMEM((TAIL,), jnp.int32),
        pltpu.VMEM((TAIL,), jnp.int32),
        pltpu.VMEM((TAIL, H), jnp.float32),
        pltpu.VMEM_SHARED((N, H), jnp.float32),
        [pltpu.SemaphoreType.DMA] * NBUF,
        [pltpu.SemaphoreType.DMA] * NBUF,
        [pltpu.SemaphoreType.DMA] * NBUF,
    ],
    compiler_params=pltpu.CompilerParams(use_tc_tiling_on_sc=False),
)


def _combine_body(h_ref, sl_ref, p_ref, out_ref):
    acc = jnp.dot(h_ref[...], sl_ref[...], preferred_element_type=jnp.float32)
    out_ref[...] = jnp.maximum(acc + p_ref[0:N, :] + p_ref[N:2 * N, :], 0.0)


def _combine(h, sl, part):
    return pl.pallas_call(
        _combine_body,
        out_shape=jax.ShapeDtypeStruct((N, H), jnp.float32),
    )(h, sl, part)


def kernel(x, bases0, wcomp0, sl0, bases1, wcomp1, sl1, edge_index, edge_type):
    src = edge_index[0]
    dst = edge_index[1]
    pad = NPLANE * 128 - R * H
    relw0 = jnp.pad(
        jnp.einsum('rb,bio->iro', wcomp0, bases0).reshape(x.shape[1], R * H),
        ((0, 0), (0, pad)))
    relw1 = jnp.pad(
        jnp.einsum('rb,bio->iro', wcomp1, bases1).reshape(H, R * H),
        ((0, 0), (0, pad)))
    zeros = jnp.zeros((N, H), jnp.float32)

    h = x
    for relw, sl in ((relw0, sl0), (relw1, sl1)):
        proj = _proj(h, relw).reshape(NPLANE * N * 4, H)
        part = _sc_edge(proj, src, dst, edge_type, zeros)
        h = _combine(h, sl, part)
    return h


# speedup vs baseline: 3.7263x; 1.0730x over previous
import functools

import jax
import jax.numpy as jnp
from jax import lax
from jax.experimental import pallas as pl
from jax.experimental.pallas import tpu as pltpu
from jax.experimental.pallas import tpu_sc as plsc

N = 10000
E = 320000
R = 51
H = 32
NB = 4

NC, NS = 2, 16
NW = NC * NS
EW = E // NW
CH = 128
NFULL = EW // CH
TAIL = EW - NFULL * CH
RPT = 624
RTAIL = N - NS * RPT


NPLANE = (R * H + 127) // 128


def _proj_body(h_ref, w_ref, out_ref):
    res = jnp.dot(h_ref[...], w_ref[...], preferred_element_type=jnp.float32)
    for j in range(NPLANE):
        out_ref[j] = res[:, j * 128:(j + 1) * 128]


def _proj(h, relw_pad):
    din = h.shape[1]
    tn = 1000
    return pl.pallas_call(
        _proj_body,
        grid=(N // tn,),
        in_specs=[
            pl.BlockSpec((tn, din), lambda t: (t, 0)),
            pl.BlockSpec((din, NPLANE * 128), lambda t: (0, 0)),
        ],
        out_specs=pl.BlockSpec((NPLANE, tn, 128), lambda t: (0, t, 0)),
        out_shape=jax.ShapeDtypeStruct((NPLANE, N, 128), jnp.float32),
    )(h, relw_pad)


NBUF = 6
NSTEP = NFULL // NBUF


LOOK = 4


def _sc_body(proj_hbm, ei_hbm, ety_hbm, zeros_hbm, out_hbm,
             src_all, ety_all, key_ring, dst_ring, rows,
             src_t, dst_t, ety_t, key_t, rows_t,
             acc_sh, sem_g, sem_d, sem_s):
    c = lax.axis_index("c")
    s = lax.axis_index("s")
    wid = s * NC + c
    ebase = wid * EW

    pltpu.sync_copy(zeros_hbm.at[pl.ds(s * RPT, RPT)],
                    acc_sh.at[pl.ds(s * RPT, RPT)])

    @pl.when(s == NS - 1)
    def _():
        pltpu.sync_copy(zeros_hbm.at[pl.ds(NS * RPT, RTAIL)],
                        acc_sh.at[pl.ds(NS * RPT, RTAIL)])

    pltpu.sync_copy(ei_hbm.at[0, pl.ds(ebase, EW)], src_all)
    pltpu.sync_copy(ety_hbm.at[pl.ds(ebase, EW)], ety_all)

    def fire(g, b):
        off = g * CH
        for k in range(CH // 16):
            sl = pl.ds(off + k * 16, 16)
            ety = ety_all[sl]
            key_ring[b, pl.ds(k * 16, 16)] = (
                (ety >> 2) * (4 * N) + src_all[sl] * 4 + (ety & 3))
        pltpu.async_copy(ei_hbm.at[1, pl.ds(ebase + off, CH)], dst_ring.at[b],
                         sem_d[b])
        pltpu.async_copy(proj_hbm.at[key_ring.at[b]], rows.at[b], sem_g[b])

    def drain_scatter(b):
        pltpu.make_async_copy(proj_hbm.at[pl.ds(0, CH)], rows.at[b],
                              sem_s[b]).wait()

    for b in range(LOOK):
        fire(b, b)

    plsc.subcore_barrier()

    def body(i, carry):
        for b in range(NBUF):
            g = i * NBUF + b
            bl = (b + LOOK) % NBUF
            if b + LOOK < NBUF:
                @pl.when(i > 0)
                def _():
                    drain_scatter(bl)
                    fire(g + LOOK, bl)

                @pl.when(i == 0)
                def _():
                    fire(g + LOOK, bl)
            else:
                @pl.when(g + LOOK < NFULL)
                def _():
                    drain_scatter(bl)
                    fire(g + LOOK, bl)
            pltpu.make_async_copy(proj_hbm.at[pl.ds(0, CH)], rows.at[b],
                                  sem_g[b]).wait()
            pltpu.make_async_copy(ei_hbm.at[1, pl.ds(0, CH)], dst_ring.at[b],
                                  sem_d[b]).wait()
            pltpu.async_copy(rows.at[b], acc_sh.at[dst_ring.at[b]], sem_s[b],
                             add=True)
        return carry

    lax.fori_loop(0, NSTEP, body, 0)
    for b in range(NBUF):
        drain_scatter(b)

    tbase = ebase + NFULL * CH
    pltpu.sync_copy(ei_hbm.at[0, pl.ds(tbase, TAIL)], src_t)
    pltpu.sync_copy(ei_hbm.at[1, pl.ds(tbase, TAIL)], dst_t)
    pltpu.sync_copy(ety_hbm.at[pl.ds(tbase, TAIL)], ety_t)
    ety = ety_t[...]
    key_t[...] = (ety >> 2) * (4 * N) + src_t[...] * 4 + (ety & 3)
    pltpu.async_copy(proj_hbm.at[key_t], rows_t, sem_g[0]).wait()
    pltpu.sync_copy(rows_t, acc_sh.at[dst_t], add=True)

    plsc.subcore_barrier()
    pltpu.sync_copy(acc_sh.at[pl.ds(s * RPT, RPT)],
                    out_hbm.at[pl.ds(c * N + s * RPT, RPT)])

    @pl.when(s == NS - 1)
    def _():
        pltpu.sync_copy(acc_sh.at[pl.ds(NS * RPT, RTAIL)],
                        out_hbm.at[pl.ds(c * N + NS * RPT, RTAIL)])


_sc_edge = pl.kernel(
    _sc_body,
    out_type=jax.ShapeDtypeStruct((NC * N, H), jnp.float32),
    mesh=plsc.VectorSubcoreMesh(core_axis_name="c", subcore_axis_name="s",
                                num_cores=NC, num_subcores=NS),
    scratch_types=[
        pltpu.VMEM((EW,), jnp.int32),
        pltpu.VMEM((EW,), jnp.int32),
        pltpu.VMEM((NBUF, CH), jnp.int32),
        pltpu.VMEM((NBUF, CH), jnp.int32),
        pltpu.VMEM((NBUF, CH, H), jnp.float32),
        pltpu.VMEM((TAIL,), jnp.int32),
        pltpu.VMEM((TAIL,), jnp.int32),
        pltpu.VMEM((TAIL,), jnp.int32),
        pltpu.VMEM((TAIL,), jnp.int32),
        pltpu.VMEM((TAIL, H), jnp.float32),
        pltpu.VMEM_SHARED((N, H), jnp.float32),
        [pltpu.SemaphoreType.DMA] * NBUF,
        [pltpu.SemaphoreType.DMA] * NBUF,
        [pltpu.SemaphoreType.DMA] * NBUF,
    ],
    compiler_params=pltpu.CompilerParams(use_tc_tiling_on_sc=False),
)


def _combine_body(h_ref, sl_ref, p_ref, out_ref):
    acc = jnp.dot(h_ref[...], sl_ref[...], preferred_element_type=jnp.float32)
    out_ref[...] = jnp.maximum(acc + p_ref[0:N, :] + p_ref[N:2 * N, :], 0.0)


def _combine(h, sl, part):
    return pl.pallas_call(
        _combine_body,
        out_shape=jax.ShapeDtypeStruct((N, H), jnp.float32),
    )(h, sl, part)


def kernel(x, bases0, wcomp0, sl0, bases1, wcomp1, sl1, edge_index, edge_type):
    pad = NPLANE * 128 - R * H
    relw0 = jnp.pad(
        jnp.einsum('rb,bio->iro', wcomp0, bases0).reshape(x.shape[1], R * H),
        ((0, 0), (0, pad)))
    relw1 = jnp.pad(
        jnp.einsum('rb,bio->iro', wcomp1, bases1).reshape(H, R * H),
        ((0, 0), (0, pad)))
    zeros = jnp.zeros((N, H), jnp.float32)

    h = x
    for relw, sl in ((relw0, sl0), (relw1, sl1)):
        proj = _proj(h, relw).reshape(NPLANE * N * 4, H)
        part = _sc_edge(proj, edge_index, edge_type, zeros)
        h = _combine(h, sl, part)
    return h
